# baseline (speedup 1.0000x reference)
import functools

import jax
import jax.numpy as jnp
from jax import lax
from jax.experimental import pallas as pl
from jax.experimental.pallas import tpu as pltpu
from jax.experimental.pallas import tpu_sc as plsc

_VOCAB = 100000
_EMBED = 25
_OUT = 32
_N = 819200

_PACK = 128 // _OUT
_VROWS = _VOCAB // _PACK
_TBLK = 1000

_W = 128
_NC = 2
_NS = 16
_NW = _NC * _NS
_BPW = _N // _NW
_K = 10
_SROWS = _K * _W
_SUPERS = _BPW // _SROWS


def _tab_body(g_ref, t_ref, o_ref):
    x = t_ref[...]
    z = x * x
    ssq = lax.dot_general(
        z, g_ref[...],
        (((1,), (0,)), ((), ())),
        preferred_element_type=jnp.float32)
    inv = lax.rsqrt(jnp.maximum(ssq, 1e-24))
    o_ref[...] = jnp.tanh(x * inv)


def _transform_table(tab_packed, gmat):
    return pl.pallas_call(
        _tab_body,
        grid=(_VROWS // _TBLK,),
        in_specs=[
            pl.BlockSpec((128, 128), lambda i: (0, 0)),
            pl.BlockSpec((_TBLK, 128), lambda i: (i, 0)),
        ],
        out_specs=pl.BlockSpec((_TBLK, 128), lambda i: (i, 0)),
        out_shape=jax.ShapeDtypeStruct((_VROWS, 128), jnp.float32),
    )(gmat, tab_packed)


def _sc_gather(tab_t, idx1d):
    mesh = plsc.VectorSubcoreMesh(core_axis_name="c", subcore_axis_name="s")

    @functools.partial(
        pl.kernel,
        out_type=jax.ShapeDtypeStruct((_N, 128), jnp.float32),
        mesh=mesh,
        compiler_params=pltpu.CompilerParams(use_tc_tiling_on_sc=False),
        scratch_types=[
            pltpu.VMEM((2, _SROWS), jnp.int32),
            pltpu.VMEM((2, _SROWS, _OUT), jnp.float32),
            pltpu.SemaphoreType.DMA,
            pltpu.SemaphoreType.DMA,
            pltpu.SemaphoreType.DMA,
            pltpu.SemaphoreType.DMA,
            pltpu.SemaphoreType.DMA,
            pltpu.SemaphoreType.DMA,
        ],
    )
    def gather_kernel(tab_hbm, idx_hbm, o_hbm, idx_v, rows_v,
                      isem0, isem1, gsem0, gsem1, wsem0, wsem1):
        isems, gsems, wsems = (isem0, isem1), (gsem0, gsem1), (wsem0, wsem1)
        wid = lax.axis_index("s") * _NC + lax.axis_index("c")
        base = wid * _BPW

        def idx_copy(b, s):
            return pltpu.make_async_copy(
                idx_hbm.at[pl.ds(base + s * _SROWS, _SROWS)],
                idx_v.at[b], isems[b])

        def out_copy(b, s):
            return pltpu.make_async_copy(
                rows_v.at[b],
                o_hbm.at[pl.ds(base + s * _SROWS, _SROWS), pl.ds(0, _OUT)],
                wsems[b])

        for b in range(2):
            idx_copy(b, b).start()

        @pl.loop(0, _SUPERS // 2)
        def _(p):
            for b in range(2):
                s = p * 2 + b
                idx_copy(b, s).wait()

                @pl.when(p >= 1)
                def _():
                    out_copy(b, s).wait()

                handles = [
                    pltpu.async_copy(
                        tab_hbm.at[idx_v.at[b, pl.ds(k * _W, _W)]],
                        rows_v.at[b, pl.ds(k * _W, _W)],
                        gsems[b])
                    for k in range(_K)
                ]
                for h in handles:
                    h.wait()

                @pl.when(p < _SUPERS // 2 - 1)
                def _():
                    idx_copy(b, s + 2).start()

                out_copy(b, s).start()

        for b in range(2):
            out_copy(b, _SUPERS - 2 + b).wait()

    return gather_kernel(tab_t, idx1d)


_SBLK = 4096


def _slice_body(w_ref, o_ref):
    o_ref[...] = w_ref[:, : _OUT]


def _slice_out(wide):
    return pl.pallas_call(
        _slice_body,
        grid=(_N // _SBLK,),
        in_specs=[pl.BlockSpec((_SBLK, 128), lambda i: (i, 0))],
        out_specs=pl.BlockSpec((_SBLK, _OUT), lambda i: (i, 0)),
        out_shape=jax.ShapeDtypeStruct((_N, _OUT), jnp.float32),
    )(wide)


def kernel(indices, table):
    idx1d = indices.astype(jnp.int32)
    tab_packed = jnp.pad(table, ((0, 0), (0, _OUT - _EMBED))).reshape(_VROWS, 128)
    lane = jnp.arange(128, dtype=jnp.int32)
    gmat = (lane[:, None] // _OUT == lane[None, :] // _OUT).astype(jnp.float32)
    tab_t = _transform_table(tab_packed, gmat).reshape(_VOCAB, _OUT)
    wide = _sc_gather(tab_t, idx1d)
    return _slice_out(wide)


# speedup vs baseline: 1.9985x; 1.9985x over previous
import functools

import jax
import jax.numpy as jnp
from jax import lax
from jax.experimental import pallas as pl
from jax.experimental.pallas import tpu as pltpu
from jax.experimental.pallas import tpu_sc as plsc

_VOCAB = 100000
_EMBED = 25
_OUT = 32
_N = 819200

_TBLK = 2000

_W = 128
_NC = 2
_NS = 16
_NW = _NC * _NS
_BPW = _N // _NW
_K = 10
_SROWS = _K * _W
_SUPERS = _BPW // _SROWS


def _tab_body(t_ref, o_ref):
    x = t_ref[...]
    ssq = jnp.sum(x * x, axis=1, keepdims=True)
    inv = lax.rsqrt(jnp.maximum(ssq, 1e-24))
    y = jnp.tanh(x * inv)
    o_ref[...] = jnp.concatenate(
        [y, jnp.zeros((_TBLK, _OUT - _EMBED), jnp.float32)], axis=1)


def _transform_table(table):
    return pl.pallas_call(
        _tab_body,
        grid=(_VOCAB // _TBLK,),
        in_specs=[pl.BlockSpec((_TBLK, _EMBED), lambda i: (i, 0))],
        out_specs=pl.BlockSpec((_TBLK, _OUT), lambda i: (i, 0)),
        out_shape=jax.ShapeDtypeStruct((_VOCAB, _OUT), jnp.float32),
    )(table)


def _sc_gather(tab_t, idx1d):
    mesh = plsc.VectorSubcoreMesh(core_axis_name="c", subcore_axis_name="s")

    @functools.partial(
        pl.kernel,
        out_type=jax.ShapeDtypeStruct((_N, 128), jnp.float32),
        mesh=mesh,
        compiler_params=pltpu.CompilerParams(use_tc_tiling_on_sc=False),
        scratch_types=[
            pltpu.VMEM((2, _SROWS), jnp.int32),
            pltpu.VMEM((2, _SROWS, _OUT), jnp.float32),
            pltpu.SemaphoreType.DMA,
            pltpu.SemaphoreType.DMA,
            pltpu.SemaphoreType.DMA,
            pltpu.SemaphoreType.DMA,
            pltpu.SemaphoreType.DMA,
            pltpu.SemaphoreType.DMA,
        ],
    )
    def gather_kernel(tab_hbm, idx_hbm, o_hbm, idx_v, rows_v,
                      isem0, isem1, gsem0, gsem1, wsem0, wsem1):
        isems, gsems, wsems = (isem0, isem1), (gsem0, gsem1), (wsem0, wsem1)
        wid = lax.axis_index("s") * _NC + lax.axis_index("c")
        base = wid * _BPW

        def idx_copy(b, s):
            return pltpu.make_async_copy(
                idx_hbm.at[pl.ds(base + s * _SROWS, _SROWS)],
                idx_v.at[b], isems[b])

        def out_copy(b, s):
            return pltpu.make_async_copy(
                rows_v.at[b],
                o_hbm.at[pl.ds(base + s * _SROWS, _SROWS), pl.ds(0, _OUT)],
                wsems[b])

        for b in range(2):
            idx_copy(b, b).start()

        @pl.loop(0, _SUPERS // 2)
        def _(p):
            for b in range(2):
                s = p * 2 + b
                idx_copy(b, s).wait()

                @pl.when(p >= 1)
                def _():
                    out_copy(b, s).wait()

                handles = [
                    pltpu.async_copy(
                        tab_hbm.at[idx_v.at[b, pl.ds(k * _W, _W)]],
                        rows_v.at[b, pl.ds(k * _W, _W)],
                        gsems[b])
                    for k in range(_K)
                ]
                for h in handles:
                    h.wait()

                @pl.when(p < _SUPERS // 2 - 1)
                def _():
                    idx_copy(b, s + 2).start()

                out_copy(b, s).start()

        for b in range(2):
            out_copy(b, _SUPERS - 2 + b).wait()

    return gather_kernel(tab_t, idx1d)


def kernel(indices, table):
    idx1d = indices.astype(jnp.int32)
    tab_t = _transform_table(table)
    return _sc_gather(tab_t, idx1d)[:, :_OUT]


# speedup vs baseline: 2.0903x; 1.0459x over previous
import functools

import jax
import jax.numpy as jnp
from jax import lax
from jax.experimental import pallas as pl
from jax.experimental.pallas import tpu as pltpu
from jax.experimental.pallas import tpu_sc as plsc

_VOCAB = 100000
_EMBED = 25
_OUT = 32
_N = 819200

_PACK = 128 // _OUT
_VROWS = _VOCAB // _PACK
_TBLK = 1000

_W = 128
_NC = 2
_NS = 16
_NW = _NC * _NS
_BPW = _N // _NW
_K = 10
_SROWS = _K * _W
_SUPERS = _BPW // _SROWS


def _tab_body(g_ref, t_ref, o_ref):
    x = t_ref[...]
    z = x * x
    ssq = lax.dot_general(
        z, g_ref[...],
        (((1,), (0,)), ((), ())),
        preferred_element_type=jnp.float32)
    inv = lax.rsqrt(jnp.maximum(ssq, 1e-24))
    o_ref[...] = jnp.tanh(x * inv)


def _transform_table(tab_packed, gmat):
    return pl.pallas_call(
        _tab_body,
        grid=(_VROWS // _TBLK,),
        in_specs=[
            pl.BlockSpec((128, 128), lambda i: (0, 0)),
            pl.BlockSpec((_TBLK, 128), lambda i: (i, 0)),
        ],
        out_specs=pl.BlockSpec((_TBLK, 128), lambda i: (i, 0)),
        out_shape=jax.ShapeDtypeStruct((_VROWS, 128), jnp.float32),
    )(gmat, tab_packed)


def _sc_gather(tab_t, idx1d):
    mesh = plsc.VectorSubcoreMesh(core_axis_name="c", subcore_axis_name="s")

    @functools.partial(
        pl.kernel,
        out_type=jax.ShapeDtypeStruct((_N, 128), jnp.float32),
        mesh=mesh,
        compiler_params=pltpu.CompilerParams(use_tc_tiling_on_sc=False),
        scratch_types=[
            pltpu.VMEM((2, _SROWS), jnp.int32),
            pltpu.VMEM((2, _SROWS, _OUT), jnp.float32),
            pltpu.SemaphoreType.DMA,
            pltpu.SemaphoreType.DMA,
            pltpu.SemaphoreType.DMA,
            pltpu.SemaphoreType.DMA,
            pltpu.SemaphoreType.DMA,
            pltpu.SemaphoreType.DMA,
        ],
    )
    def gather_kernel(tab_hbm, idx_hbm, o_hbm, idx_v, rows_v,
                      isem0, isem1, gsem0, gsem1, wsem0, wsem1):
        isems, gsems, wsems = (isem0, isem1), (gsem0, gsem1), (wsem0, wsem1)
        wid = lax.axis_index("s") * _NC + lax.axis_index("c")
        base = wid * _BPW

        def idx_copy(b, s):
            return pltpu.make_async_copy(
                idx_hbm.at[pl.ds(base + s * _SROWS, _SROWS)],
                idx_v.at[b], isems[b])

        def out_copy(b, s):
            return pltpu.make_async_copy(
                rows_v.at[b],
                o_hbm.at[pl.ds(base + s * _SROWS, _SROWS), pl.ds(0, _OUT)],
                wsems[b])

        for b in range(2):
            idx_copy(b, b).start()

        @pl.loop(0, _SUPERS // 2)
        def _(p):
            for b in range(2):
                s = p * 2 + b
                idx_copy(b, s).wait()

                @pl.when(p >= 1)
                def _():
                    out_copy(b, s).wait()

                handles = [
                    pltpu.async_copy(
                        tab_hbm.at[idx_v.at[b, pl.ds(k * _W, _W)]],
                        rows_v.at[b, pl.ds(k * _W, _W)],
                        gsems[b])
                    for k in range(_K)
                ]
                for h in handles:
                    h.wait()

                @pl.when(p < _SUPERS // 2 - 1)
                def _():
                    idx_copy(b, s + 2).start()

                out_copy(b, s).start()

        for b in range(2):
            out_copy(b, _SUPERS - 2 + b).wait()

    return gather_kernel(tab_t, idx1d)


def kernel(indices, table):
    idx1d = indices.astype(jnp.int32)
    tab_packed = jnp.pad(table, ((0, 0), (0, _OUT - _EMBED))).reshape(_VROWS, 128)
    lane = jnp.arange(128, dtype=jnp.int32)
    gmat = (lane[:, None] // _OUT == lane[None, :] // _OUT).astype(jnp.float32)
    tab_t = _transform_table(tab_packed, gmat).reshape(_VOCAB, _OUT)
    return _sc_gather(tab_t, idx1d)[:, :_OUT]
